# Initial kernel scaffold; baseline (speedup 1.0000x reference)
import functools

import jax
import jax.numpy as jnp
from jax import lax
from jax.experimental import pallas as pl
from jax.experimental.pallas import tpu as pltpu
from jax.experimental.pallas import tpu_sc as plsc

N = 50000
E = 800000
H = 96
P = 6
W16 = 16
TILES = 16
NCORES = 2
ROWS_PER_TILE = N // TILES
PARTS_PER_CORE = P // NCORES

AGG_EDGES_PER_TILE = E // TILES
AGG_CHUNK = 80
AGG_ITERS = AGG_EDGES_PER_TILE // AGG_CHUNK

DEG_EDGES_PER_TILE = E // (NCORES * TILES)
DEG_CHUNK = 40
DEG_ITERS = DEG_EDGES_PER_TILE // DEG_CHUNK

_mesh = plsc.VectorSubcoreMesh(core_axis_name="c", subcore_axis_name="s")


def _leaky(x):
    return jnp.where(x > 0, x, 0.01 * x)


@functools.partial(
    pl.kernel,
    out_type=jax.ShapeDtypeStruct((NCORES, N, W16), jnp.float32),
    mesh=_mesh,
    scratch_types=[
        pltpu.VMEM((DEG_CHUNK,), jnp.int32),
        pltpu.VMEM((DEG_CHUNK, W16), jnp.float32),
        pltpu.VMEM_SHARED((N, W16), jnp.float32),
    ],
)
def _deg_kernel(dst_hbm, ones_hbm, zeros_hbm, out_hbm, didx, ones_v, acc):
    sc = lax.axis_index("c")
    sub = lax.axis_index("s")
    row0 = sub * ROWS_PER_TILE
    pltpu.sync_copy(zeros_hbm, acc.at[pl.ds(row0, ROWS_PER_TILE)])
    pltpu.sync_copy(ones_hbm, ones_v)
    plsc.subcore_barrier()
    base = sc * (E // NCORES) + sub * DEG_EDGES_PER_TILE

    def body(i, carry):
        off = base + i * DEG_CHUNK
        pltpu.sync_copy(dst_hbm.at[pl.ds(off, DEG_CHUNK)], didx)
        pltpu.sync_copy(ones_v, acc.at[didx], add=True)
        return carry

    lax.fori_loop(0, DEG_ITERS, body, 0)
    plsc.subcore_barrier()
    pltpu.sync_copy(acc.at[pl.ds(row0, ROWS_PER_TILE)],
                    out_hbm.at[sc, pl.ds(row0, ROWS_PER_TILE)])


@functools.partial(
    pl.kernel,
    out_type=jax.ShapeDtypeStruct((N, P, W16), jnp.float32),
    mesh=_mesh,
    scratch_types=[
        pltpu.VMEM((AGG_CHUNK,), jnp.int32),
        pltpu.VMEM((AGG_CHUNK,), jnp.int32),
        pltpu.VMEM((AGG_CHUNK,), jnp.int32),
        pltpu.VMEM((AGG_CHUNK, W16), jnp.float32),
        pltpu.VMEM_SHARED((N, W16), jnp.float32),
        pltpu.SemaphoreType.DMA,
    ],
)
def _agg_kernel(utab_hbm, src_hbm, dst_hbm, zeros_hbm, out_hbm,
                sidx, didx, gidx, rows, acc, sem):
    sc = lax.axis_index("c")
    sub = lax.axis_index("s")
    row0 = sub * ROWS_PER_TILE
    for pp in range(PARTS_PER_CORE):
        part = sc * PARTS_PER_CORE + pp
        pltpu.sync_copy(zeros_hbm, acc.at[pl.ds(row0, ROWS_PER_TILE)])
        plsc.subcore_barrier()

        def body(i, carry):
            off = sub * AGG_EDGES_PER_TILE + i * AGG_CHUNK
            pltpu.sync_copy(src_hbm.at[pl.ds(off, AGG_CHUNK)], sidx)
            pltpu.sync_copy(dst_hbm.at[pl.ds(off, AGG_CHUNK)], didx)
            for j in range(AGG_CHUNK // 16):
                s = sidx[pl.ds(j * 16, 16)]
                gidx[pl.ds(j * 16, 16)] = s * P + part
            pltpu.async_copy(utab_hbm.at[gidx], rows, sem).wait()
            pltpu.sync_copy(rows, acc.at[didx], add=True)
            return carry

        lax.fori_loop(0, AGG_ITERS, body, 0)
        plsc.subcore_barrier()
        pltpu.sync_copy(acc.at[pl.ds(row0, ROWS_PER_TILE)],
                        out_hbm.at[pl.ds(row0, ROWS_PER_TILE), part])
        plsc.subcore_barrier()


BR = 1000
GRID = N // BR


def _full(shape):
    return pl.BlockSpec(shape, lambda i: tuple(0 for _ in shape))


def _rows(shape):
    return pl.BlockSpec(shape, lambda i: (i,) + tuple(0 for _ in shape[1:]))


def _embed_pre1_body(dx, cx, degp, wbd, bc, wg0, bg0, wg1,
                     xdc_o, dis_o, u1_o):
    xd = dx[:, 6:32]
    xc = jnp.dot(cx[...], wbd[...], preferred_element_type=jnp.float32) + bc[...]
    xdc = jnp.concatenate([xd, xc], axis=1)
    xdc_o[...] = xdc
    deg = degp[0, :, 0] + degp[1, :, 0] + 1.0
    dis = lax.rsqrt(deg)[:, None]
    dis_o[...] = dis
    xg0 = _leaky(jnp.dot(xdc, wg0[...], preferred_element_type=jnp.float32)
                 + bg0[...])
    u1_o[...] = jnp.dot(xg0, wg1[...], preferred_element_type=jnp.float32) * dis


def _embed_pre1(dx, cx, degp, wbd, bc, wg0, bg0, wg1):
    return pl.pallas_call(
        _embed_pre1_body,
        grid=(GRID,),
        in_specs=[
            _rows((BR, 32)),
            _rows((BR, 48)),
            pl.BlockSpec((NCORES, BR, W16), lambda i: (0, i, 0)),
            _full((48, 12)),
            _full((1, 12)),
            _full((38, H)),
            _full((1, H)),
            _full((H, H)),
        ],
        out_specs=[_rows((BR, 38)), _rows((BR, 1)), _rows((BR, H))],
        out_shape=[
            jax.ShapeDtypeStruct((N, 38), jnp.float32),
            jax.ShapeDtypeStruct((N, 1), jnp.float32),
            jax.ShapeDtypeStruct((N, H), jnp.float32),
        ],
    )(dx, cx, degp, wbd, bc, wg0, bg0, wg1)


def _pre2_body(acc1, u1, dis, b1, wg2, u2_o):
    d = dis[...]
    xg1 = _leaky(d * (acc1[...] + u1[...]) + b1[...])
    u2_o[...] = jnp.dot(xg1, wg2[...], preferred_element_type=jnp.float32) * d


def _pre2(acc1, u1, dis, b1, wg2):
    return pl.pallas_call(
        _pre2_body,
        grid=(GRID,),
        in_specs=[
            _rows((BR, H)),
            _rows((BR, H)),
            _rows((BR, 1)),
            _full((1, H)),
            _full((H, H)),
        ],
        out_specs=_rows((BR, H)),
        out_shape=jax.ShapeDtypeStruct((N, H), jnp.float32),
    )(acc1, u1, dis, b1, wg2)


def _head_body(acc2, u2, dis, b2, xdc, wd1, bd1, wd2, bd2, wcr, bcr,
               wp1, bp1, wp2, bp2, out_o):
    d = dis[...]
    xg2 = _leaky(d * (acc2[...] + u2[...]) + b2[...])
    x = jnp.concatenate([xdc[...], xg2], axis=1)
    h = _leaky(jnp.dot(x, wd1[...], preferred_element_type=jnp.float32)
               + bd1[...])
    deep = _leaky(jnp.dot(h, wd2[...], preferred_element_type=jnp.float32)
                  + bd2[...])
    xl = x
    for i in range(2):
        s = jnp.sum(xl * wcr[i:i + 1, :], axis=1, keepdims=True)
        xl = x * s + bcr[i:i + 1, :] + xl
    xc2 = jnp.concatenate([deep, xl], axis=1)
    p1 = _leaky(jnp.dot(xc2, wp1[...], preferred_element_type=jnp.float32)
                + bp1[...])
    p2 = jnp.dot(p1, wp2[...], preferred_element_type=jnp.float32) + bp2[...]
    out_o[...] = jax.nn.sigmoid(p2)


def _head(acc2, u2, dis, b2, xdc, wd1, bd1, wd2, bd2, wcr, bcr,
          wp1, bp1, wp2, bp2):
    NH1 = 134
    return pl.pallas_call(
        _head_body,
        grid=(GRID,),
        in_specs=[
            _rows((BR, H)),
            _rows((BR, H)),
            _rows((BR, 1)),
            _full((1, H)),
            _rows((BR, 38)),
            _full((NH1, NH1)),
            _full((1, NH1)),
            _full((NH1, NH1)),
            _full((1, NH1)),
            _full((2, NH1)),
            _full((2, NH1)),
            _full((2 * NH1, NH1)),
            _full((1, NH1)),
            _full((NH1, 1)),
            _full((1, 1)),
        ],
        out_specs=_rows((BR, 1)),
        out_shape=jax.ShapeDtypeStruct((N, 1), jnp.float32),
    )(acc2, u2, dis, b2, xdc, wd1, bd1, wd2, bd2, wcr, bcr,
      wp1, bp1, wp2, bp2)


def kernel(discrete_x, continous_x, edge_index, edge_attr, churn_date,
           W_c, b_c, W_g0, b_g0, W_gcn1, b_gcn1, W_gcn2, b_gcn2,
           W_d1, b_d1, W_d2, b_d2, w_cross, b_cross, W_p1, b_p1, W_p2, b_p2):
    src = edge_index[0]
    dst = edge_index[1]
    zeros = jnp.zeros((ROWS_PER_TILE, W16), jnp.float32)
    ones = jnp.ones((DEG_CHUNK, W16), jnp.float32)

    degp = _deg_kernel(dst, ones, zeros)

    wbd = jnp.zeros((48, 12), jnp.float32)
    for g in range(3):
        wbd = wbd.at[g * 16:(g + 1) * 16, g * 4:(g + 1) * 4].set(W_c)
    bc = jnp.tile(b_c, 3)[None, :]

    xdc, dis, u1 = _embed_pre1(discrete_x, continous_x, degp, wbd, bc,
                               W_g0, b_g0[None, :], W_gcn1)
    acc1 = _agg_kernel(u1.reshape(N * P, W16), src, dst, zeros)
    acc1 = acc1.reshape(N, H)
    u2 = _pre2(acc1, u1, dis, b_gcn1[None, :], W_gcn2)
    acc2 = _agg_kernel(u2.reshape(N * P, W16), src, dst, zeros)
    acc2 = acc2.reshape(N, H)
    return _head(acc2, u2, dis, b_gcn2[None, :], xdc,
                 W_d1, b_d1[None, :], W_d2, b_d2[None, :],
                 w_cross, b_cross, W_p1, b_p1[None, :], W_p2, b_p2[None, :])


# speedup vs baseline: 3.1463x; 3.1463x over previous
import functools

import jax
import jax.numpy as jnp
from jax import lax
from jax.experimental import pallas as pl
from jax.experimental.pallas import tpu as pltpu
from jax.experimental.pallas import tpu_sc as plsc

N = 50000
E = 800000
H = 96
P = 6
W16 = 16
LANES = 128
GROUPS = LANES // W16
TILES = 16
NCORES = 2
NPAD = 51200
RPT = NPAD // TILES
PARTS_PER_CORE = P // NCORES

AGG_EDGES_PER_TILE = E // TILES
AGG_CHUNK = 80
AGG_ITERS = AGG_EDGES_PER_TILE // AGG_CHUNK

DEG_EDGES_PER_TILE = E // (NCORES * TILES)
DEG_CHUNK = 40
DEG_ITERS = DEG_EDGES_PER_TILE // DEG_CHUNK

_mesh = plsc.VectorSubcoreMesh(core_axis_name="c", subcore_axis_name="s",
                               num_cores=NCORES, num_subcores=TILES)
_sc_params = pltpu.CompilerParams(use_tc_tiling_on_sc=False)


def _leaky(x):
    return jnp.where(x > 0, x, 0.01 * x)


@functools.partial(
    pl.kernel,
    out_type=jax.ShapeDtypeStruct((NPAD, GROUPS, W16), jnp.float32),
    mesh=_mesh,
    scratch_types=[
        pltpu.VMEM((DEG_CHUNK,), jnp.int32),
        pltpu.VMEM((DEG_CHUNK, W16), jnp.float32),
        pltpu.VMEM_SHARED((NPAD, W16), jnp.float32),
    ],
    compiler_params=_sc_params,
)
def _deg_kernel(dst_hbm, ones_hbm, zeros_hbm, out_hbm, didx, ones_v, acc):
    sc = lax.axis_index("c")
    sub = lax.axis_index("s")
    row0 = sub * RPT
    pltpu.sync_copy(zeros_hbm, acc.at[pl.ds(row0, RPT)])
    pltpu.sync_copy(ones_hbm, ones_v)
    plsc.subcore_barrier()
    base = sc * (E // NCORES) + sub * DEG_EDGES_PER_TILE

    def body(i, carry):
        off = base + i * DEG_CHUNK
        pltpu.sync_copy(dst_hbm.at[pl.ds(off, DEG_CHUNK)], didx)
        pltpu.sync_copy(ones_v, acc.at[didx], add=True)
        return carry

    lax.fori_loop(0, DEG_ITERS, body, 0)
    plsc.subcore_barrier()
    pltpu.sync_copy(acc.at[pl.ds(row0, RPT)],
                    out_hbm.at[pl.ds(row0, RPT), sc])


@functools.partial(
    pl.kernel,
    out_type=jax.ShapeDtypeStruct((NPAD, GROUPS, W16), jnp.float32),
    mesh=_mesh,
    scratch_types=[
        pltpu.VMEM((AGG_CHUNK,), jnp.int32),
        pltpu.VMEM((AGG_CHUNK,), jnp.int32),
        pltpu.VMEM((AGG_CHUNK,), jnp.int32),
        pltpu.VMEM((AGG_CHUNK, W16), jnp.float32),
        pltpu.VMEM_SHARED((NPAD, W16), jnp.float32),
        pltpu.SemaphoreType.DMA,
    ],
    compiler_params=_sc_params,
)
def _agg_kernel(utab_hbm, src_hbm, dst_hbm, zeros_hbm, out_hbm,
                sidx, didx, gidx, rows, acc, sem):
    sc = lax.axis_index("c")
    sub = lax.axis_index("s")
    row0 = sub * RPT
    for pp in range(PARTS_PER_CORE):
        part = sc * PARTS_PER_CORE + pp
        pltpu.sync_copy(zeros_hbm, acc.at[pl.ds(row0, RPT)])
        plsc.subcore_barrier()

        def body(i, carry):
            off = sub * AGG_EDGES_PER_TILE + i * AGG_CHUNK
            pltpu.sync_copy(src_hbm.at[pl.ds(off, AGG_CHUNK)], sidx)
            pltpu.sync_copy(dst_hbm.at[pl.ds(off, AGG_CHUNK)], didx)
            for j in range(AGG_CHUNK // 16):
                s = sidx[pl.ds(j * 16, 16)]
                gidx[pl.ds(j * 16, 16)] = s * GROUPS + part
            pltpu.async_copy(utab_hbm.at[gidx], rows, sem).wait()
            pltpu.sync_copy(rows, acc.at[didx], add=True)
            return carry

        lax.fori_loop(0, AGG_ITERS, body, 0)
        plsc.subcore_barrier()
        pltpu.sync_copy(acc.at[pl.ds(row0, RPT)],
                        out_hbm.at[pl.ds(row0, RPT), part])
        plsc.subcore_barrier()


BR = 1000
GRID = N // BR


def _full(shape):
    return pl.BlockSpec(shape, lambda i: tuple(0 for _ in shape))


def _rows(shape):
    return pl.BlockSpec(shape, lambda i: (i,) + tuple(0 for _ in shape[1:]))


def _with_dis(u, dis):
    pad = jnp.zeros((u.shape[0], LANES - H - 1), jnp.float32)
    return jnp.concatenate([u, dis, pad], axis=1)


def _embed_pre1_body(dx, cx, degp, wbd, bc, wg0, bg0, wg1, xdc_o, u1_o):
    xd = dx[:, 6:32]
    xc = jnp.dot(cx[...], wbd[...], preferred_element_type=jnp.float32) + bc[...]
    xdc = jnp.concatenate([xd, xc], axis=1)
    xdc_o[...] = xdc
    deg = degp[:, 0] + degp[:, W16] + 1.0
    dis = lax.rsqrt(deg)[:, None]
    xg0 = _leaky(jnp.dot(xdc, wg0[...], preferred_element_type=jnp.float32)
                 + bg0[...])
    u1 = jnp.dot(xg0, wg1[...], preferred_element_type=jnp.float32) * dis
    u1_o[...] = _with_dis(u1, dis)


def _embed_pre1(dx, cx, degp, wbd, bc, wg0, bg0, wg1):
    return pl.pallas_call(
        _embed_pre1_body,
        grid=(GRID,),
        in_specs=[
            _rows((BR, 32)),
            _rows((BR, 48)),
            _rows((BR, LANES)),
            _full((48, 12)),
            _full((1, 12)),
            _full((38, H)),
            _full((1, H)),
            _full((H, H)),
        ],
        out_specs=[_rows((BR, 38)), _rows((BR, LANES))],
        out_shape=[
            jax.ShapeDtypeStruct((N, 38), jnp.float32),
            jax.ShapeDtypeStruct((N, LANES), jnp.float32),
        ],
    )(dx, cx, degp, wbd, bc, wg0, bg0, wg1)


def _pre2_body(acc1, u1d, b1, wg2, u2_o):
    d = u1d[:, H:H + 1]
    xg1 = _leaky(d * (acc1[:, :H] + u1d[:, :H]) + b1[...])
    u2 = jnp.dot(xg1, wg2[...], preferred_element_type=jnp.float32) * d
    u2_o[...] = _with_dis(u2, d)


def _pre2(acc1, u1d, b1, wg2):
    return pl.pallas_call(
        _pre2_body,
        grid=(GRID,),
        in_specs=[
            _rows((BR, LANES)),
            _rows((BR, LANES)),
            _full((1, H)),
            _full((H, H)),
        ],
        out_specs=_rows((BR, LANES)),
        out_shape=jax.ShapeDtypeStruct((N, LANES), jnp.float32),
    )(acc1, u1d, b1, wg2)


def _head_body(acc2, u2d, b2, xdc, wd1, bd1, wd2, bd2, wcr, bcr,
               wp1, bp1, wp2, bp2, out_o):
    d = u2d[:, H:H + 1]
    xg2 = _leaky(d * (acc2[:, :H] + u2d[:, :H]) + b2[...])
    x = jnp.concatenate([xdc[...], xg2], axis=1)
    h = _leaky(jnp.dot(x, wd1[...], preferred_element_type=jnp.float32)
               + bd1[...])
    deep = _leaky(jnp.dot(h, wd2[...], preferred_element_type=jnp.float32)
                  + bd2[...])
    xl = x
    for i in range(2):
        s = jnp.sum(xl * wcr[i:i + 1, :], axis=1, keepdims=True)
        xl = x * s + bcr[i:i + 1, :] + xl
    xc2 = jnp.concatenate([deep, xl], axis=1)
    p1 = _leaky(jnp.dot(xc2, wp1[...], preferred_element_type=jnp.float32)
                + bp1[...])
    p2 = jnp.dot(p1, wp2[...], preferred_element_type=jnp.float32) + bp2[...]
    out_o[...] = jax.nn.sigmoid(p2)


def _head(acc2, u2d, b2, xdc, wd1, bd1, wd2, bd2, wcr, bcr,
          wp1, bp1, wp2, bp2):
    NH1 = 134
    return pl.pallas_call(
        _head_body,
        grid=(GRID,),
        in_specs=[
            _rows((BR, LANES)),
            _rows((BR, LANES)),
            _full((1, H)),
            _rows((BR, 38)),
            _full((NH1, NH1)),
            _full((1, NH1)),
            _full((NH1, NH1)),
            _full((1, NH1)),
            _full((2, NH1)),
            _full((2, NH1)),
            _full((2 * NH1, NH1)),
            _full((1, NH1)),
            _full((NH1, 1)),
            _full((1, 1)),
        ],
        out_specs=_rows((BR, 1)),
        out_shape=jax.ShapeDtypeStruct((N, 1), jnp.float32),
    )(acc2, u2d, b2, xdc, wd1, bd1, wd2, bd2, wcr, bcr,
      wp1, bp1, wp2, bp2)


def kernel(discrete_x, continous_x, edge_index, edge_attr, churn_date,
           W_c, b_c, W_g0, b_g0, W_gcn1, b_gcn1, W_gcn2, b_gcn2,
           W_d1, b_d1, W_d2, b_d2, w_cross, b_cross, W_p1, b_p1, W_p2, b_p2):
    src = edge_index[0]
    dst = edge_index[1]
    zeros = jnp.zeros((RPT, W16), jnp.float32)
    ones = jnp.ones((DEG_CHUNK, W16), jnp.float32)

    degp = _deg_kernel(dst, ones, zeros).reshape(NPAD, LANES)[:N]

    wbd = jnp.zeros((48, 12), jnp.float32)
    for g in range(3):
        wbd = wbd.at[g * 16:(g + 1) * 16, g * 4:(g + 1) * 4].set(W_c)
    bc = jnp.tile(b_c, 3)[None, :]

    xdc, u1d = _embed_pre1(discrete_x, continous_x, degp, wbd, bc,
                           W_g0, b_g0[None, :], W_gcn1)
    acc1 = _agg_kernel(u1d.reshape(N * GROUPS, W16), src, dst, zeros)
    acc1 = acc1.reshape(NPAD, LANES)[:N]
    u2d = _pre2(acc1, u1d, b_gcn1[None, :], W_gcn2)
    acc2 = _agg_kernel(u2d.reshape(N * GROUPS, W16), src, dst, zeros)
    acc2 = acc2.reshape(NPAD, LANES)[:N]
    return _head(acc2, u2d, b_gcn2[None, :], xdc,
                 W_d1, b_d1[None, :], W_d2, b_d2[None, :],
                 w_cross, b_cross, W_p1, b_p1[None, :], W_p2, b_p2[None, :])


# speedup vs baseline: 10.2228x; 3.2491x over previous
import functools

import jax
import jax.numpy as jnp
from jax import lax
from jax.experimental import pallas as pl
from jax.experimental.pallas import tpu as pltpu
from jax.experimental.pallas import tpu_sc as plsc

N = 50000
E = 800000
H = 96
P = 6
W16 = 16
LANES = 128
GROUPS = LANES // W16
TILES = 16
NCORES = 2
NPAD = 51200
RPT = NPAD // TILES
PARTS_PER_CORE = P // NCORES

AGG_EDGES_PER_TILE = E // TILES
AGG_CHUNK = 80
AGG_CROWS = AGG_EDGES_PER_TILE // AGG_CHUNK
GK = 5
NGRP = AGG_CROWS // GK

DEG_EDGES_PER_TILE = E // (NCORES * TILES)
DEG_CHUNK = 40
DEG_ITERS = DEG_EDGES_PER_TILE // DEG_CHUNK

_mesh = plsc.VectorSubcoreMesh(core_axis_name="c", subcore_axis_name="s",
                               num_cores=NCORES, num_subcores=TILES)
_sc_params = pltpu.CompilerParams(use_tc_tiling_on_sc=False)


def _leaky(x):
    return jnp.where(x > 0, x, 0.01 * x)


@functools.partial(
    pl.kernel,
    out_type=jax.ShapeDtypeStruct((NPAD, GROUPS, W16), jnp.float32),
    mesh=_mesh,
    scratch_types=[
        pltpu.VMEM((DEG_CHUNK,), jnp.int32),
        pltpu.VMEM((DEG_CHUNK, W16), jnp.float32),
        pltpu.VMEM_SHARED((NPAD, W16), jnp.float32),
    ],
    compiler_params=_sc_params,
)
def _deg_kernel(dst_hbm, ones_hbm, zeros_hbm, out_hbm, didx, ones_v, acc):
    sc = lax.axis_index("c")
    sub = lax.axis_index("s")
    row0 = sub * RPT
    pltpu.sync_copy(zeros_hbm, acc.at[pl.ds(row0, RPT)])
    pltpu.sync_copy(ones_hbm, ones_v)
    plsc.subcore_barrier()
    base = sc * (E // NCORES) + sub * DEG_EDGES_PER_TILE

    def body(i, carry):
        off = base + i * DEG_CHUNK
        pltpu.sync_copy(dst_hbm.at[pl.ds(off, DEG_CHUNK)], didx)
        pltpu.sync_copy(ones_v, acc.at[didx], add=True)
        return carry

    lax.fori_loop(0, DEG_ITERS, body, 0)
    plsc.subcore_barrier()
    pltpu.sync_copy(acc.at[pl.ds(row0, RPT)],
                    out_hbm.at[pl.ds(row0, RPT), sc])


@functools.partial(
    pl.kernel,
    out_type=jax.ShapeDtypeStruct((NPAD, GROUPS, W16), jnp.float32),
    mesh=_mesh,
    scratch_types=[
        pltpu.VMEM((AGG_CROWS, AGG_CHUNK), jnp.int32),
        pltpu.VMEM((GK, AGG_CHUNK), jnp.int32),
        pltpu.VMEM((GK, AGG_CHUNK), jnp.int32),
        pltpu.VMEM((GK, AGG_CHUNK, W16), jnp.float32),
        pltpu.VMEM((GK, AGG_CHUNK, W16), jnp.float32),
        pltpu.VMEM_SHARED((NPAD, W16), jnp.float32),
        pltpu.SemaphoreType.DMA,
        pltpu.SemaphoreType.DMA,
        pltpu.SemaphoreType.DMA,
        pltpu.SemaphoreType.DMA,
        pltpu.SemaphoreType.DMA,
        pltpu.SemaphoreType.DMA,
    ],
    compiler_params=_sc_params,
)
def _agg_kernel(utab_hbm, gtab_hbm, dst2d_hbm, zeros_hbm, out_hbm,
                didx, gidxA, gidxB, rowsA, rowsB, acc,
                sgA, sgB, ssA, ssB, siA, siB):
    sc = lax.axis_index("c")
    sub = lax.axis_index("s")
    row0 = sub * RPT
    crow0 = sub * AGG_CROWS
    pltpu.sync_copy(dst2d_hbm.at[pl.ds(crow0, AGG_CROWS)], didx)

    def fire_gathers(gidx, rows, sem):
        for k in range(GK):
            pltpu.async_copy(utab_hbm.at[gidx.at[k]], rows.at[k], sem)

    def wait_gathers(gidx, rows, sem):
        for k in range(GK):
            pltpu.make_async_copy(
                utab_hbm.at[gidx.at[k]], rows.at[k], sem).wait()

    def fire_scatters(rows, g, sem):
        for k in range(GK):
            pltpu.async_copy(rows.at[k], acc.at[didx.at[g * GK + k]], sem,
                             add=True)

    def wait_scatters(rows, g, sem):
        for k in range(GK):
            pltpu.make_async_copy(
                rows.at[k], acc.at[didx.at[g * GK + k]], sem).wait()

    def gidx_src(part, g):
        return gtab_hbm.at[part, pl.ds(crow0 + g * GK, GK)]

    for pp in range(PARTS_PER_CORE):
        part = sc * PARTS_PER_CORE + pp
        pltpu.sync_copy(zeros_hbm, acc.at[pl.ds(row0, RPT)])
        plsc.subcore_barrier()

        sets = ((gidxA, rowsA, sgA, ssA, siA), (gidxB, rowsB, sgB, ssB, siB))

        pltpu.sync_copy(gidx_src(part, 0), gidxA)
        fire_gathers(gidxA, rowsA, sgA)
        pltpu.async_copy(gidx_src(part, 1), gidxB, siB)

        def phase(g, cur, nxt, first=False, fire_next=True, load_next=True):
            cg, cr, csg, css, csi = cur
            ng, nr, nsg, nss, nsi = nxt
            if not first:
                wait_scatters(nr, g - 1, nss)
            if fire_next:
                pltpu.make_async_copy(gidx_src(part, g + 1), ng, nsi).wait()
                fire_gathers(ng, nr, nsg)
            wait_gathers(cg, cr, csg)
            if load_next:
                pltpu.async_copy(gidx_src(part, g + 2), cg, csi)
            fire_scatters(cr, g, css)

        phase(0, sets[0], sets[1], first=True)

        def body(i, carry):
            g = 1 + 2 * i
            phase(g, sets[1], sets[0])
            phase(g + 1, sets[0], sets[1])
            return carry

        lax.fori_loop(0, (NGRP - 3) // 2, body, 0)
        phase(NGRP - 2, sets[1], sets[0], load_next=False)
        phase(NGRP - 1, sets[0], sets[1], fire_next=False, load_next=False)
        wait_scatters(rowsA, NGRP - 1, ssA)

        plsc.subcore_barrier()
        pltpu.sync_copy(acc.at[pl.ds(row0, RPT)],
                        out_hbm.at[pl.ds(row0, RPT), part])
        plsc.subcore_barrier()


BR = 1000
GRID = N // BR


def _full(shape):
    return pl.BlockSpec(shape, lambda i: tuple(0 for _ in shape))


def _rows(shape):
    return pl.BlockSpec(shape, lambda i: (i,) + tuple(0 for _ in shape[1:]))


def _with_dis(u, dis):
    pad = jnp.zeros((u.shape[0], LANES - H - 1), jnp.float32)
    return jnp.concatenate([u, dis, pad], axis=1)


def _gtab_body(s2, out_o):
    parts = lax.broadcasted_iota(jnp.int32, (P, 1, 1), 0)
    out_o[...] = s2[...][None, :, :] * GROUPS + parts


def _gtab(src):
    ER = E // LANES
    s2 = src.reshape(ER, LANES)
    out = pl.pallas_call(
        _gtab_body,
        in_specs=[pl.BlockSpec((ER, LANES), lambda: (0, 0))],
        out_specs=pl.BlockSpec((P, ER, LANES), lambda: (0, 0, 0)),
        out_shape=jax.ShapeDtypeStruct((P, ER, LANES), jnp.int32),
    )(s2)
    return out.reshape(P, E // AGG_CHUNK, AGG_CHUNK)


def _embed_pre1_body(dx, cx, degp, wbd, bc, wg0, bg0, wg1, xdc_o, u1_o):
    xd = dx[:, 6:32]
    xc = jnp.dot(cx[...], wbd[...], preferred_element_type=jnp.float32) + bc[...]
    xdc = jnp.concatenate([xd, xc], axis=1)
    xdc_o[...] = xdc
    deg = degp[:, 0] + degp[:, W16] + 1.0
    dis = lax.rsqrt(deg)[:, None]
    xg0 = _leaky(jnp.dot(xdc, wg0[...], preferred_element_type=jnp.float32)
                 + bg0[...])
    u1 = jnp.dot(xg0, wg1[...], preferred_element_type=jnp.float32) * dis
    u1_o[...] = _with_dis(u1, dis)


def _embed_pre1(dx, cx, degp, wbd, bc, wg0, bg0, wg1):
    return pl.pallas_call(
        _embed_pre1_body,
        grid=(GRID,),
        in_specs=[
            _rows((BR, 32)),
            _rows((BR, 48)),
            _rows((BR, LANES)),
            _full((48, 12)),
            _full((1, 12)),
            _full((38, H)),
            _full((1, H)),
            _full((H, H)),
        ],
        out_specs=[_rows((BR, 38)), _rows((BR, LANES))],
        out_shape=[
            jax.ShapeDtypeStruct((N, 38), jnp.float32),
            jax.ShapeDtypeStruct((N, LANES), jnp.float32),
        ],
    )(dx, cx, degp, wbd, bc, wg0, bg0, wg1)


def _pre2_body(acc1, u1d, b1, wg2, u2_o):
    d = u1d[:, H:H + 1]
    xg1 = _leaky(d * (acc1[:, :H] + u1d[:, :H]) + b1[...])
    u2 = jnp.dot(xg1, wg2[...], preferred_element_type=jnp.float32) * d
    u2_o[...] = _with_dis(u2, d)


def _pre2(acc1, u1d, b1, wg2):
    return pl.pallas_call(
        _pre2_body,
        grid=(GRID,),
        in_specs=[
            _rows((BR, LANES)),
            _rows((BR, LANES)),
            _full((1, H)),
            _full((H, H)),
        ],
        out_specs=_rows((BR, LANES)),
        out_shape=jax.ShapeDtypeStruct((N, LANES), jnp.float32),
    )(acc1, u1d, b1, wg2)


def _head_body(acc2, u2d, b2, xdc, wd1, bd1, wd2, bd2, wcr, bcr,
               wp1, bp1, wp2, bp2, out_o):
    d = u2d[:, H:H + 1]
    xg2 = _leaky(d * (acc2[:, :H] + u2d[:, :H]) + b2[...])
    x = jnp.concatenate([xdc[...], xg2], axis=1)
    h = _leaky(jnp.dot(x, wd1[...], preferred_element_type=jnp.float32)
               + bd1[...])
    deep = _leaky(jnp.dot(h, wd2[...], preferred_element_type=jnp.float32)
                  + bd2[...])
    xl = x
    for i in range(2):
        s = jnp.sum(xl * wcr[i:i + 1, :], axis=1, keepdims=True)
        xl = x * s + bcr[i:i + 1, :] + xl
    xc2 = jnp.concatenate([deep, xl], axis=1)
    p1 = _leaky(jnp.dot(xc2, wp1[...], preferred_element_type=jnp.float32)
                + bp1[...])
    p2 = jnp.dot(p1, wp2[...], preferred_element_type=jnp.float32) + bp2[...]
    out_o[...] = jax.nn.sigmoid(p2)


def _head(acc2, u2d, b2, xdc, wd1, bd1, wd2, bd2, wcr, bcr,
          wp1, bp1, wp2, bp2):
    NH1 = 134
    return pl.pallas_call(
        _head_body,
        grid=(GRID,),
        in_specs=[
            _rows((BR, LANES)),
            _rows((BR, LANES)),
            _full((1, H)),
            _rows((BR, 38)),
            _full((NH1, NH1)),
            _full((1, NH1)),
            _full((NH1, NH1)),
            _full((1, NH1)),
            _full((2, NH1)),
            _full((2, NH1)),
            _full((2 * NH1, NH1)),
            _full((1, NH1)),
            _full((NH1, 1)),
            _full((1, 1)),
        ],
        out_specs=_rows((BR, 1)),
        out_shape=jax.ShapeDtypeStruct((N, 1), jnp.float32),
    )(acc2, u2d, b2, xdc, wd1, bd1, wd2, bd2, wcr, bcr,
      wp1, bp1, wp2, bp2)


def kernel(discrete_x, continous_x, edge_index, edge_attr, churn_date,
           W_c, b_c, W_g0, b_g0, W_gcn1, b_gcn1, W_gcn2, b_gcn2,
           W_d1, b_d1, W_d2, b_d2, w_cross, b_cross, W_p1, b_p1, W_p2, b_p2):
    src = edge_index[0]
    dst = edge_index[1]
    zeros = jnp.zeros((RPT, W16), jnp.float32)
    ones = jnp.ones((DEG_CHUNK, W16), jnp.float32)

    degp = _deg_kernel(dst, ones, zeros).reshape(NPAD, LANES)[:N]

    wbd = jnp.zeros((48, 12), jnp.float32)
    for g in range(3):
        wbd = wbd.at[g * 16:(g + 1) * 16, g * 4:(g + 1) * 4].set(W_c)
    bc = jnp.tile(b_c, 3)[None, :]

    gtab = _gtab(src)
    dst2d = dst.reshape(E // AGG_CHUNK, AGG_CHUNK)
    xdc, u1d = _embed_pre1(discrete_x, continous_x, degp, wbd, bc,
                           W_g0, b_g0[None, :], W_gcn1)
    acc1 = _agg_kernel(u1d.reshape(N * GROUPS, W16), gtab, dst2d, zeros)
    acc1 = acc1.reshape(NPAD, LANES)[:N]
    u2d = _pre2(acc1, u1d, b_gcn1[None, :], W_gcn2)
    acc2 = _agg_kernel(u2d.reshape(N * GROUPS, W16), gtab, dst2d, zeros)
    acc2 = acc2.reshape(NPAD, LANES)[:N]
    return _head(acc2, u2d, b_gcn2[None, :], xdc,
                 W_d1, b_d1[None, :], W_d2, b_d2[None, :],
                 w_cross, b_cross, W_p1, b_p1[None, :], W_p2, b_p2[None, :])


# speedup vs baseline: 12.2686x; 1.2001x over previous
import functools

import jax
import jax.numpy as jnp
from jax import lax
from jax.experimental import pallas as pl
from jax.experimental.pallas import tpu as pltpu
from jax.experimental.pallas import tpu_sc as plsc

N = 50000
E = 800000
H = 96
P = 6
W16 = 16
LANES = 128
GROUPS = LANES // W16
TILES = 16
NCORES = 2
NPAD = 51200
RPT = NPAD // TILES
PARTS_PER_CORE = P // NCORES

AGG_EDGES_PER_TILE = E // TILES
AGG_CHUNK = 80
AGG_CROWS = AGG_EDGES_PER_TILE // AGG_CHUNK
GK = 5
NGRP = AGG_CROWS // GK

DEG_EDGES_PER_TILE = E // (NCORES * TILES)
DEG_CHUNK = 40
DEG_ITERS = DEG_EDGES_PER_TILE // DEG_CHUNK

_mesh = plsc.VectorSubcoreMesh(core_axis_name="c", subcore_axis_name="s",
                               num_cores=NCORES, num_subcores=TILES)
_sc_params = pltpu.CompilerParams(use_tc_tiling_on_sc=False)


def _leaky(x):
    return jnp.where(x > 0, x, 0.01 * x)


@functools.partial(
    pl.kernel,
    out_type=jax.ShapeDtypeStruct((NPAD, GROUPS, W16), jnp.float32),
    mesh=_mesh,
    scratch_types=[
        pltpu.VMEM((DEG_CHUNK,), jnp.int32),
        pltpu.VMEM((DEG_CHUNK, W16), jnp.float32),
        pltpu.VMEM_SHARED((NPAD, W16), jnp.float32),
    ],
    compiler_params=_sc_params,
)
def _deg_kernel(dst_hbm, ones_hbm, zeros_hbm, out_hbm, didx, ones_v, acc):
    sc = lax.axis_index("c")
    sub = lax.axis_index("s")
    row0 = sub * RPT
    pltpu.sync_copy(zeros_hbm, acc.at[pl.ds(row0, RPT)])
    pltpu.sync_copy(ones_hbm, ones_v)
    plsc.subcore_barrier()
    base = sc * (E // NCORES) + sub * DEG_EDGES_PER_TILE

    def body(i, carry):
        off = base + i * DEG_CHUNK
        pltpu.sync_copy(dst_hbm.at[pl.ds(off, DEG_CHUNK)], didx)
        pltpu.sync_copy(ones_v, acc.at[didx], add=True)
        return carry

    lax.fori_loop(0, DEG_ITERS, body, 0)
    plsc.subcore_barrier()
    pltpu.sync_copy(acc.at[pl.ds(row0, RPT)],
                    out_hbm.at[pl.ds(row0, RPT), sc])


@functools.partial(
    pl.kernel,
    out_type=jax.ShapeDtypeStruct((NPAD, GROUPS, W16), jnp.float32),
    mesh=_mesh,
    scratch_types=[
        pltpu.VMEM((AGG_CROWS, AGG_CHUNK), jnp.int32),
        pltpu.VMEM((GK * AGG_CHUNK,), jnp.int32),
        pltpu.VMEM((GK * AGG_CHUNK,), jnp.int32),
        pltpu.VMEM((GK, AGG_CHUNK, W16), jnp.float32),
        pltpu.VMEM((GK, AGG_CHUNK, W16), jnp.float32),
        pltpu.VMEM_SHARED((NPAD, W16), jnp.float32),
        pltpu.SemaphoreType.DMA,
        pltpu.SemaphoreType.DMA,
        pltpu.SemaphoreType.DMA,
        pltpu.SemaphoreType.DMA,
        pltpu.SemaphoreType.DMA,
        pltpu.SemaphoreType.DMA,
    ],
    compiler_params=_sc_params,
)
def _agg_kernel(utab_hbm, src_hbm, dst2d_hbm, zeros_hbm, out_hbm,
                didx, gidxA, gidxB, rowsA, rowsB, acc,
                sgA, sgB, ssA, ssB, siA, siB):
    sc = lax.axis_index("c")
    sub = lax.axis_index("s")
    row0 = sub * RPT
    crow0 = sub * AGG_CROWS

    pltpu.sync_copy(dst2d_hbm.at[pl.ds(crow0, AGG_CROWS)], didx)

    def to_gidx(gidx, part):
        for j in range(GK * AGG_CHUNK // W16):
            s = gidx[pl.ds(j * W16, W16)]
            gidx[pl.ds(j * W16, W16)] = s * GROUPS + part

    def fire_gathers(gidx, rows, sem):
        for k in range(GK):
            pltpu.async_copy(
                utab_hbm.at[gidx.at[pl.ds(k * AGG_CHUNK, AGG_CHUNK)]],
                rows.at[k], sem)

    def wait_gathers(gidx, rows, sem):
        for k in range(GK):
            pltpu.make_async_copy(
                utab_hbm.at[gidx.at[pl.ds(k * AGG_CHUNK, AGG_CHUNK)]],
                rows.at[k], sem).wait()

    def fire_scatters(rows, g, sem):
        for k in range(GK):
            pltpu.async_copy(rows.at[k], acc.at[didx.at[g * GK + k]], sem,
                             add=True)

    def wait_scatters(rows, g, sem):
        for k in range(GK):
            pltpu.make_async_copy(
                rows.at[k], acc.at[didx.at[g * GK + k]], sem).wait()

    def gidx_src(part, g):
        return src_hbm.at[pl.ds((crow0 + g * GK) * AGG_CHUNK, GK * AGG_CHUNK)]

    for pp in range(PARTS_PER_CORE):
        part = sc * PARTS_PER_CORE + pp
        pltpu.sync_copy(zeros_hbm, acc.at[pl.ds(row0, RPT)])
        plsc.subcore_barrier()

        sets = ((gidxA, rowsA, sgA, ssA, siA), (gidxB, rowsB, sgB, ssB, siB))

        pltpu.sync_copy(gidx_src(part, 0), gidxA)
        to_gidx(gidxA, part)
        fire_gathers(gidxA, rowsA, sgA)
        pltpu.async_copy(gidx_src(part, 1), gidxB, siB)

        def phase(g, cur, nxt, first=False, fire_next=True, load_next=True):
            cg, cr, csg, css, csi = cur
            ng, nr, nsg, nss, nsi = nxt
            if not first:
                wait_scatters(nr, g - 1, nss)
            if fire_next:
                pltpu.make_async_copy(gidx_src(part, g + 1), ng, nsi).wait()
                to_gidx(ng, part)
                fire_gathers(ng, nr, nsg)
            wait_gathers(cg, cr, csg)
            if load_next:
                pltpu.async_copy(gidx_src(part, g + 2), cg, csi)
            fire_scatters(cr, g, css)

        phase(0, sets[0], sets[1], first=True)

        def body(i, carry):
            g = 1 + 2 * i
            phase(g, sets[1], sets[0])
            phase(g + 1, sets[0], sets[1])
            return carry

        lax.fori_loop(0, (NGRP - 3) // 2, body, 0)
        phase(NGRP - 2, sets[1], sets[0], load_next=False)
        phase(NGRP - 1, sets[0], sets[1], fire_next=False, load_next=False)
        wait_scatters(rowsA, NGRP - 1, ssA)

        plsc.subcore_barrier()
        pltpu.sync_copy(acc.at[pl.ds(row0, RPT)],
                        out_hbm.at[pl.ds(row0, RPT), part])
        plsc.subcore_barrier()


BR = 1000
GRID = N // BR


def _full(shape):
    return pl.BlockSpec(shape, lambda i: tuple(0 for _ in shape))


def _rows(shape):
    return pl.BlockSpec(shape, lambda i: (i,) + tuple(0 for _ in shape[1:]))


def _with_dis(u, dis):
    pad = jnp.zeros((u.shape[0], LANES - H - 1), jnp.float32)
    return jnp.concatenate([u, dis, pad], axis=1)


def _embed_pre1_body(dx, cx, degp, wbd, bc, wg0, bg0, wg1, xdc_o, u1_o):
    xd = dx[:, 6:32]
    xc = jnp.dot(cx[...], wbd[...], preferred_element_type=jnp.float32) + bc[...]
    xdc = jnp.concatenate([xd, xc], axis=1)
    xdc_o[...] = xdc
    deg = degp[:, 0] + degp[:, W16] + 1.0
    dis = lax.rsqrt(deg)[:, None]
    xg0 = _leaky(jnp.dot(xdc, wg0[...], preferred_element_type=jnp.float32)
                 + bg0[...])
    u1 = jnp.dot(xg0, wg1[...], preferred_element_type=jnp.float32) * dis
    u1_o[...] = _with_dis(u1, dis)


def _embed_pre1(dx, cx, degp, wbd, bc, wg0, bg0, wg1):
    return pl.pallas_call(
        _embed_pre1_body,
        grid=(GRID,),
        in_specs=[
            _rows((BR, 32)),
            _rows((BR, 48)),
            _rows((BR, LANES)),
            _full((48, 12)),
            _full((1, 12)),
            _full((38, H)),
            _full((1, H)),
            _full((H, H)),
        ],
        out_specs=[_rows((BR, 38)), _rows((BR, LANES))],
        out_shape=[
            jax.ShapeDtypeStruct((N, 38), jnp.float32),
            jax.ShapeDtypeStruct((N, LANES), jnp.float32),
        ],
    )(dx, cx, degp, wbd, bc, wg0, bg0, wg1)


def _pre2_body(acc1, u1d, b1, wg2, u2_o):
    d = u1d[:, H:H + 1]
    xg1 = _leaky(d * (acc1[:, :H] + u1d[:, :H]) + b1[...])
    u2 = jnp.dot(xg1, wg2[...], preferred_element_type=jnp.float32) * d
    u2_o[...] = _with_dis(u2, d)


def _pre2(acc1, u1d, b1, wg2):
    return pl.pallas_call(
        _pre2_body,
        grid=(GRID,),
        in_specs=[
            _rows((BR, LANES)),
            _rows((BR, LANES)),
            _full((1, H)),
            _full((H, H)),
        ],
        out_specs=_rows((BR, LANES)),
        out_shape=jax.ShapeDtypeStruct((N, LANES), jnp.float32),
    )(acc1, u1d, b1, wg2)


def _head_body(acc2, u2d, b2, xdc, wd1, bd1, wd2, bd2, wcr, bcr,
               wp1, bp1, wp2, bp2, out_o):
    d = u2d[:, H:H + 1]
    xg2 = _leaky(d * (acc2[:, :H] + u2d[:, :H]) + b2[...])
    x = jnp.concatenate([xdc[...], xg2], axis=1)
    h = _leaky(jnp.dot(x, wd1[...], preferred_element_type=jnp.float32)
               + bd1[...])
    deep = _leaky(jnp.dot(h, wd2[...], preferred_element_type=jnp.float32)
                  + bd2[...])
    xl = x
    for i in range(2):
        s = jnp.sum(xl * wcr[i:i + 1, :], axis=1, keepdims=True)
        xl = x * s + bcr[i:i + 1, :] + xl
    xc2 = jnp.concatenate([deep, xl], axis=1)
    p1 = _leaky(jnp.dot(xc2, wp1[...], preferred_element_type=jnp.float32)
                + bp1[...])
    p2 = jnp.dot(p1, wp2[...], preferred_element_type=jnp.float32) + bp2[...]
    out_o[...] = jax.nn.sigmoid(p2)


def _head(acc2, u2d, b2, xdc, wd1, bd1, wd2, bd2, wcr, bcr,
          wp1, bp1, wp2, bp2):
    NH1 = 134
    return pl.pallas_call(
        _head_body,
        grid=(GRID,),
        in_specs=[
            _rows((BR, LANES)),
            _rows((BR, LANES)),
            _full((1, H)),
            _rows((BR, 38)),
            _full((NH1, NH1)),
            _full((1, NH1)),
            _full((NH1, NH1)),
            _full((1, NH1)),
            _full((2, NH1)),
            _full((2, NH1)),
            _full((2 * NH1, NH1)),
            _full((1, NH1)),
            _full((NH1, 1)),
            _full((1, 1)),
        ],
        out_specs=_rows((BR, 1)),
        out_shape=jax.ShapeDtypeStruct((N, 1), jnp.float32),
    )(acc2, u2d, b2, xdc, wd1, bd1, wd2, bd2, wcr, bcr,
      wp1, bp1, wp2, bp2)


def kernel(discrete_x, continous_x, edge_index, edge_attr, churn_date,
           W_c, b_c, W_g0, b_g0, W_gcn1, b_gcn1, W_gcn2, b_gcn2,
           W_d1, b_d1, W_d2, b_d2, w_cross, b_cross, W_p1, b_p1, W_p2, b_p2):
    src = edge_index[0]
    dst = edge_index[1]
    zeros = jnp.zeros((RPT, W16), jnp.float32)
    ones = jnp.ones((DEG_CHUNK, W16), jnp.float32)

    degp = _deg_kernel(dst, ones, zeros).reshape(NPAD, LANES)

    wbd = jnp.zeros((48, 12), jnp.float32)
    for g in range(3):
        wbd = wbd.at[g * 16:(g + 1) * 16, g * 4:(g + 1) * 4].set(W_c)
    bc = jnp.tile(b_c, 3)[None, :]

    dst2d = dst.reshape(E // AGG_CHUNK, AGG_CHUNK)
    xdc, u1d = _embed_pre1(discrete_x, continous_x, degp, wbd, bc,
                           W_g0, b_g0[None, :], W_gcn1)
    acc1 = _agg_kernel(u1d.reshape(N * GROUPS, W16), src,
                       dst2d, zeros).reshape(NPAD, LANES)
    u2d = _pre2(acc1, u1d, b_gcn1[None, :], W_gcn2)
    acc2 = _agg_kernel(u2d.reshape(N * GROUPS, W16), src,
                       dst2d, zeros).reshape(NPAD, LANES)
    return _head(acc2, u2d, b_gcn2[None, :], xdc,
                 W_d1, b_d1[None, :], W_d2, b_d2[None, :],
                 w_cross, b_cross, W_p1, b_p1[None, :], W_p2, b_p2[None, :])


# speedup vs baseline: 14.2040x; 1.1578x over previous
import functools

import jax
import jax.numpy as jnp
from jax import lax
from jax.experimental import pallas as pl
from jax.experimental.pallas import tpu as pltpu
from jax.experimental.pallas import tpu_sc as plsc

N = 50000
E = 800000
H = 96
P = 6
W16 = 16
LANES = 128
GROUPS = LANES // W16
TILES = 16
NCORES = 2
NPAD = 51200
RPT = NPAD // TILES
PARTS_PER_CORE = P // NCORES

AGG_EDGES_PER_TILE = E // TILES
AGG_CHUNK = 80
AGG_CROWS = AGG_EDGES_PER_TILE // AGG_CHUNK
GK = 5
NGRP = AGG_CROWS // GK

DEG_EDGES_PER_TILE = E // (NCORES * TILES)
DEG_CHUNK = 100
DEG_CROWS = DEG_EDGES_PER_TILE // DEG_CHUNK
DEG_FPB = 10

_mesh = plsc.VectorSubcoreMesh(core_axis_name="c", subcore_axis_name="s",
                               num_cores=NCORES, num_subcores=TILES)
_sc_params = pltpu.CompilerParams(use_tc_tiling_on_sc=False)


def _leaky(x):
    return jnp.where(x > 0, x, 0.01 * x)


@functools.partial(
    pl.kernel,
    out_type=jax.ShapeDtypeStruct((NPAD, GROUPS, W16), jnp.float32),
    mesh=_mesh,
    scratch_types=[
        pltpu.VMEM((DEG_CROWS, DEG_CHUNK), jnp.int32),
        pltpu.VMEM((DEG_CHUNK, W16), jnp.float32),
        pltpu.VMEM_SHARED((NPAD, W16), jnp.float32),
        pltpu.SemaphoreType.DMA,
    ],
    compiler_params=_sc_params,
)
def _deg_kernel(dst2d_hbm, ones_hbm, zeros_hbm, out_hbm, didx, ones_v, acc,
                sem):
    sc = lax.axis_index("c")
    sub = lax.axis_index("s")
    row0 = sub * RPT
    crow0 = (sc * TILES + sub) * DEG_CROWS
    pltpu.sync_copy(dst2d_hbm.at[pl.ds(crow0, DEG_CROWS)], didx)
    pltpu.sync_copy(zeros_hbm, acc.at[pl.ds(row0, RPT)])
    pltpu.sync_copy(ones_hbm, ones_v)
    plsc.subcore_barrier()

    def fire(i, carry):
        for k in range(DEG_FPB):
            pltpu.async_copy(ones_v, acc.at[didx.at[i * DEG_FPB + k]], sem,
                             add=True)
        return carry

    def drain(i, carry):
        for k in range(DEG_FPB):
            pltpu.make_async_copy(
                ones_v, acc.at[didx.at[i * DEG_FPB + k]], sem).wait()
        return carry

    lax.fori_loop(0, DEG_CROWS // DEG_FPB, fire, 0)
    lax.fori_loop(0, DEG_CROWS // DEG_FPB, drain, 0)
    plsc.subcore_barrier()
    pltpu.sync_copy(acc.at[pl.ds(row0, RPT)],
                    out_hbm.at[pl.ds(row0, RPT), sc])


@functools.partial(
    pl.kernel,
    out_type=jax.ShapeDtypeStruct((NPAD, GROUPS, W16), jnp.float32),
    mesh=_mesh,
    scratch_types=[
        pltpu.VMEM((AGG_CROWS, AGG_CHUNK), jnp.int32),
        pltpu.VMEM((GK * AGG_CHUNK,), jnp.int32),
        pltpu.VMEM((GK * AGG_CHUNK,), jnp.int32),
        pltpu.VMEM((GK, AGG_CHUNK, W16), jnp.float32),
        pltpu.VMEM((GK, AGG_CHUNK, W16), jnp.float32),
        pltpu.VMEM_SHARED((NPAD, W16), jnp.float32),
        pltpu.SemaphoreType.DMA,
        pltpu.SemaphoreType.DMA,
        pltpu.SemaphoreType.DMA,
        pltpu.SemaphoreType.DMA,
        pltpu.SemaphoreType.DMA,
        pltpu.SemaphoreType.DMA,
    ],
    compiler_params=_sc_params,
)
def _agg_kernel(utab_hbm, src_hbm, dst2d_hbm, zeros_hbm, out_hbm,
                didx, gidxA, gidxB, rowsA, rowsB, acc,
                sgA, sgB, ssA, ssB, siA, siB):
    sc = lax.axis_index("c")
    sub = lax.axis_index("s")
    row0 = sub * RPT
    crow0 = sub * AGG_CROWS

    pltpu.sync_copy(dst2d_hbm.at[pl.ds(crow0, AGG_CROWS)], didx)

    def to_gidx(gidx, part):
        for j in range(GK * AGG_CHUNK // W16):
            s = gidx[pl.ds(j * W16, W16)]
            gidx[pl.ds(j * W16, W16)] = s * GROUPS + part

    def fire_gathers(gidx, rows, sem):
        for k in range(GK):
            pltpu.async_copy(
                utab_hbm.at[gidx.at[pl.ds(k * AGG_CHUNK, AGG_CHUNK)]],
                rows.at[k], sem)

    def wait_gathers(gidx, rows, sem):
        for k in range(GK):
            pltpu.make_async_copy(
                utab_hbm.at[gidx.at[pl.ds(k * AGG_CHUNK, AGG_CHUNK)]],
                rows.at[k], sem).wait()

    def fire_scatters(rows, g, sem):
        for k in range(GK):
            pltpu.async_copy(rows.at[k], acc.at[didx.at[g * GK + k]], sem,
                             add=True)

    def wait_scatters(rows, g, sem):
        for k in range(GK):
            pltpu.make_async_copy(
                rows.at[k], acc.at[didx.at[g * GK + k]], sem).wait()

    def gidx_src(part, g):
        return src_hbm.at[pl.ds((crow0 + g * GK) * AGG_CHUNK, GK * AGG_CHUNK)]

    for pp in range(PARTS_PER_CORE):
        part = sc * PARTS_PER_CORE + pp
        pltpu.sync_copy(zeros_hbm, acc.at[pl.ds(row0, RPT)])
        plsc.subcore_barrier()

        sets = ((gidxA, rowsA, sgA, ssA, siA), (gidxB, rowsB, sgB, ssB, siB))

        pltpu.sync_copy(gidx_src(part, 0), gidxA)
        to_gidx(gidxA, part)
        fire_gathers(gidxA, rowsA, sgA)
        pltpu.async_copy(gidx_src(part, 1), gidxB, siB)

        def phase(g, cur, nxt, first=False, fire_next=True, load_next=True):
            cg, cr, csg, css, csi = cur
            ng, nr, nsg, nss, nsi = nxt
            if not first:
                wait_scatters(nr, g - 1, nss)
            if fire_next:
                pltpu.make_async_copy(gidx_src(part, g + 1), ng, nsi).wait()
                to_gidx(ng, part)
                fire_gathers(ng, nr, nsg)
            wait_gathers(cg, cr, csg)
            if load_next:
                pltpu.async_copy(gidx_src(part, g + 2), cg, csi)
            fire_scatters(cr, g, css)

        phase(0, sets[0], sets[1], first=True)

        def body(i, carry):
            g = 1 + 2 * i
            phase(g, sets[1], sets[0])
            phase(g + 1, sets[0], sets[1])
            return carry

        lax.fori_loop(0, (NGRP - 3) // 2, body, 0)
        phase(NGRP - 2, sets[1], sets[0], load_next=False)
        phase(NGRP - 1, sets[0], sets[1], fire_next=False, load_next=False)
        wait_scatters(rowsA, NGRP - 1, ssA)

        plsc.subcore_barrier()
        pltpu.sync_copy(acc.at[pl.ds(row0, RPT)],
                        out_hbm.at[pl.ds(row0, RPT), part])
        plsc.subcore_barrier()


BR = 1000
GRID = N // BR


def _full(shape):
    return pl.BlockSpec(shape, lambda i: tuple(0 for _ in shape))


def _rows(shape):
    return pl.BlockSpec(shape, lambda i: (i,) + tuple(0 for _ in shape[1:]))


def _with_dis(u, dis):
    pad = jnp.zeros((u.shape[0], LANES - H - 1), jnp.float32)
    return jnp.concatenate([u, dis, pad], axis=1)


def _embed_pre1_body(dx, cx, degp, wbd, bc, wg0, bg0, wg1, xdc_o, u1_o):
    xd = dx[:, 6:32]
    xc = jnp.dot(cx[...], wbd[...], preferred_element_type=jnp.float32) + bc[...]
    xdc = jnp.concatenate([xd, xc], axis=1)
    xdc_o[...] = xdc
    deg = degp[:, 0] + degp[:, W16] + 1.0
    dis = lax.rsqrt(deg)[:, None]
    xg0 = _leaky(jnp.dot(xdc, wg0[...], preferred_element_type=jnp.float32)
                 + bg0[...])
    u1 = jnp.dot(xg0, wg1[...], preferred_element_type=jnp.float32) * dis
    u1_o[...] = _with_dis(u1, dis)


def _embed_pre1(dx, cx, degp, wbd, bc, wg0, bg0, wg1):
    return pl.pallas_call(
        _embed_pre1_body,
        grid=(GRID,),
        in_specs=[
            _rows((BR, 32)),
            _rows((BR, 48)),
            _rows((BR, LANES)),
            _full((48, 12)),
            _full((1, 12)),
            _full((38, H)),
            _full((1, H)),
            _full((H, H)),
        ],
        out_specs=[_rows((BR, 38)), _rows((BR, LANES))],
        out_shape=[
            jax.ShapeDtypeStruct((N, 38), jnp.float32),
            jax.ShapeDtypeStruct((N, LANES), jnp.float32),
        ],
    )(dx, cx, degp, wbd, bc, wg0, bg0, wg1)


def _pre2_body(acc1, u1d, b1, wg2, u2_o):
    d = u1d[:, H:H + 1]
    xg1 = _leaky(d * (acc1[:, :H] + u1d[:, :H]) + b1[...])
    u2 = jnp.dot(xg1, wg2[...], preferred_element_type=jnp.float32) * d
    u2_o[...] = _with_dis(u2, d)


def _pre2(acc1, u1d, b1, wg2):
    return pl.pallas_call(
        _pre2_body,
        grid=(GRID,),
        in_specs=[
            _rows((BR, LANES)),
            _rows((BR, LANES)),
            _full((1, H)),
            _full((H, H)),
        ],
        out_specs=_rows((BR, LANES)),
        out_shape=jax.ShapeDtypeStruct((N, LANES), jnp.float32),
    )(acc1, u1d, b1, wg2)


def _head_body(acc2, u2d, b2, xdc, wd1, bd1, wd2, bd2, wcr, bcr,
               wp1, bp1, wp2, bp2, out_o):
    d = u2d[:, H:H + 1]
    xg2 = _leaky(d * (acc2[:, :H] + u2d[:, :H]) + b2[...])
    x = jnp.concatenate([xdc[...], xg2], axis=1)
    h = _leaky(jnp.dot(x, wd1[...], preferred_element_type=jnp.float32)
               + bd1[...])
    deep = _leaky(jnp.dot(h, wd2[...], preferred_element_type=jnp.float32)
                  + bd2[...])
    xl = x
    for i in range(2):
        s = jnp.sum(xl * wcr[i:i + 1, :], axis=1, keepdims=True)
        xl = x * s + bcr[i:i + 1, :] + xl
    xc2 = jnp.concatenate([deep, xl], axis=1)
    p1 = _leaky(jnp.dot(xc2, wp1[...], preferred_element_type=jnp.float32)
                + bp1[...])
    p2 = jnp.dot(p1, wp2[...], preferred_element_type=jnp.float32) + bp2[...]
    out_o[...] = jax.nn.sigmoid(p2)


def _head(acc2, u2d, b2, xdc, wd1, bd1, wd2, bd2, wcr, bcr,
          wp1, bp1, wp2, bp2):
    NH1 = 134
    return pl.pallas_call(
        _head_body,
        grid=(GRID,),
        in_specs=[
            _rows((BR, LANES)),
            _rows((BR, LANES)),
            _full((1, H)),
            _rows((BR, 38)),
            _full((NH1, NH1)),
            _full((1, NH1)),
            _full((NH1, NH1)),
            _full((1, NH1)),
            _full((2, NH1)),
            _full((2, NH1)),
            _full((2 * NH1, NH1)),
            _full((1, NH1)),
            _full((NH1, 1)),
            _full((1, 1)),
        ],
        out_specs=_rows((BR, 1)),
        out_shape=jax.ShapeDtypeStruct((N, 1), jnp.float32),
    )(acc2, u2d, b2, xdc, wd1, bd1, wd2, bd2, wcr, bcr,
      wp1, bp1, wp2, bp2)


def kernel(discrete_x, continous_x, edge_index, edge_attr, churn_date,
           W_c, b_c, W_g0, b_g0, W_gcn1, b_gcn1, W_gcn2, b_gcn2,
           W_d1, b_d1, W_d2, b_d2, w_cross, b_cross, W_p1, b_p1, W_p2, b_p2):
    src = edge_index[0]
    dst = edge_index[1]
    zeros = jnp.zeros((RPT, W16), jnp.float32)
    ones = jnp.ones((DEG_CHUNK, W16), jnp.float32)

    degp = _deg_kernel(dst.reshape(E // DEG_CHUNK, DEG_CHUNK),
                       ones, zeros).reshape(NPAD, LANES)

    wbd = jnp.zeros((48, 12), jnp.float32)
    for g in range(3):
        wbd = wbd.at[g * 16:(g + 1) * 16, g * 4:(g + 1) * 4].set(W_c)
    bc = jnp.tile(b_c, 3)[None, :]

    dst2d = dst.reshape(E // AGG_CHUNK, AGG_CHUNK)
    xdc, u1d = _embed_pre1(discrete_x, continous_x, degp, wbd, bc,
                           W_g0, b_g0[None, :], W_gcn1)
    acc1 = _agg_kernel(u1d.reshape(N * GROUPS, W16), src,
                       dst2d, zeros).reshape(NPAD, LANES)
    u2d = _pre2(acc1, u1d, b_gcn1[None, :], W_gcn2)
    acc2 = _agg_kernel(u2d.reshape(N * GROUPS, W16), src,
                       dst2d, zeros).reshape(NPAD, LANES)
    return _head(acc2, u2d, b_gcn2[None, :], xdc,
                 W_d1, b_d1[None, :], W_d2, b_d2[None, :],
                 w_cross, b_cross, W_p1, b_p1[None, :], W_p2, b_p2[None, :])


# speedup vs baseline: 14.3640x; 1.0113x over previous
import functools

import jax
import jax.numpy as jnp
from jax import lax
from jax.experimental import pallas as pl
from jax.experimental.pallas import tpu as pltpu
from jax.experimental.pallas import tpu_sc as plsc

N = 50000
E = 800000
H = 96
P = 6
W16 = 16
LANES = 128
GROUPS = LANES // W16
TILES = 16
NCORES = 2
NPAD = 51200
RPT = NPAD // TILES
PARTS_PER_CORE = P // NCORES

AGG_EDGES_PER_TILE = E // TILES
AGG_CHUNK = 80
AGG_CROWS = AGG_EDGES_PER_TILE // AGG_CHUNK
GK = 5
NGRP = AGG_CROWS // GK

DEG_EDGES_PER_TILE = E // (NCORES * TILES)
DEG_CHUNK = 100
DEG_CROWS = DEG_EDGES_PER_TILE // DEG_CHUNK
DEG_FPB = 10

_mesh = plsc.VectorSubcoreMesh(core_axis_name="c", subcore_axis_name="s",
                               num_cores=NCORES, num_subcores=TILES)
_sc_params = pltpu.CompilerParams(use_tc_tiling_on_sc=False)


def _leaky(x):
    return jnp.where(x > 0, x, 0.01 * x)


@functools.partial(
    pl.kernel,
    out_type=jax.ShapeDtypeStruct((NPAD, GROUPS, W16), jnp.float32),
    mesh=_mesh,
    scratch_types=[
        pltpu.VMEM((DEG_CROWS, DEG_CHUNK), jnp.int32),
        pltpu.VMEM((DEG_CHUNK, W16), jnp.float32),
        pltpu.VMEM_SHARED((NPAD, W16), jnp.float32),
        pltpu.SemaphoreType.DMA,
    ],
    compiler_params=_sc_params,
)
def _deg_kernel(dst2d_hbm, ones_hbm, zeros_hbm, out_hbm, didx, ones_v, acc,
                sem):
    sc = lax.axis_index("c")
    sub = lax.axis_index("s")
    row0 = sub * RPT
    crow0 = (sc * TILES + sub) * DEG_CROWS
    pltpu.sync_copy(dst2d_hbm.at[pl.ds(crow0, DEG_CROWS)], didx)
    pltpu.sync_copy(zeros_hbm, acc.at[pl.ds(row0, RPT)])
    pltpu.sync_copy(ones_hbm, ones_v)
    plsc.subcore_barrier()

    def fire(i, carry):
        for k in range(DEG_FPB):
            pltpu.async_copy(ones_v, acc.at[didx.at[i * DEG_FPB + k]], sem,
                             add=True)
        return carry

    def drain(i, carry):
        for k in range(DEG_FPB):
            pltpu.make_async_copy(
                ones_v, acc.at[didx.at[i * DEG_FPB + k]], sem).wait()
        return carry

    lax.fori_loop(0, DEG_CROWS // DEG_FPB, fire, 0)
    lax.fori_loop(0, DEG_CROWS // DEG_FPB, drain, 0)
    plsc.subcore_barrier()
    pltpu.sync_copy(acc.at[pl.ds(row0, RPT)],
                    out_hbm.at[pl.ds(row0, RPT), sc])


@functools.partial(
    pl.kernel,
    out_type=jax.ShapeDtypeStruct((NPAD, GROUPS, W16), jnp.float32),
    mesh=_mesh,
    scratch_types=[
        pltpu.VMEM((AGG_CROWS, AGG_CHUNK), jnp.int32),
        pltpu.VMEM((GK * AGG_CHUNK,), jnp.int32),
        pltpu.VMEM((GK * AGG_CHUNK,), jnp.int32),
        pltpu.VMEM((GK, AGG_CHUNK, W16), jnp.float32),
        pltpu.VMEM((GK, AGG_CHUNK, W16), jnp.float32),
        pltpu.VMEM_SHARED((NPAD, W16), jnp.float32),
        pltpu.SemaphoreType.DMA,
        pltpu.SemaphoreType.DMA,
        pltpu.SemaphoreType.DMA,
        pltpu.SemaphoreType.DMA,
        pltpu.SemaphoreType.DMA,
        pltpu.SemaphoreType.DMA,
    ],
    compiler_params=_sc_params,
)
def _agg_kernel(utab_hbm, src_hbm, dst2d_hbm, zeros_hbm, out_hbm,
                didx, gidxA, gidxB, rowsA, rowsB, acc,
                sgA, sgB, ssA, ssB, siA, siB):
    sc = lax.axis_index("c")
    sub = lax.axis_index("s")
    row0 = sub * RPT
    crow0 = sub * AGG_CROWS

    pltpu.sync_copy(dst2d_hbm.at[pl.ds(crow0, AGG_CROWS)], didx)

    def to_gidx(gidx, part):
        for j in range(GK * AGG_CHUNK // W16):
            s = gidx[pl.ds(j * W16, W16)]
            gidx[pl.ds(j * W16, W16)] = s * GROUPS + part

    def fire_gathers(gidx, rows, sem):
        for k in range(GK):
            pltpu.async_copy(
                utab_hbm.at[gidx.at[pl.ds(k * AGG_CHUNK, AGG_CHUNK)]],
                rows.at[k], sem)

    def wait_gathers(gidx, rows, sem):
        for k in range(GK):
            pltpu.make_async_copy(
                utab_hbm.at[gidx.at[pl.ds(k * AGG_CHUNK, AGG_CHUNK)]],
                rows.at[k], sem).wait()

    def fire_scatters(rows, g, sem):
        for k in range(GK):
            pltpu.async_copy(rows.at[k], acc.at[didx.at[g * GK + k]], sem,
                             add=True)

    def wait_scatters(rows, g, sem):
        for k in range(GK):
            pltpu.make_async_copy(
                rows.at[k], acc.at[didx.at[g * GK + k]], sem).wait()

    def gidx_src(part, g):
        return src_hbm.at[pl.ds((crow0 + g * GK) * AGG_CHUNK, GK * AGG_CHUNK)]

    for pp in range(PARTS_PER_CORE):
        part = sc * PARTS_PER_CORE + pp
        pltpu.sync_copy(zeros_hbm, acc.at[pl.ds(row0, RPT)])
        plsc.subcore_barrier()

        sets = ((gidxA, rowsA, sgA, ssA, siA), (gidxB, rowsB, sgB, ssB, siB))

        pltpu.sync_copy(gidx_src(part, 0), gidxA)
        to_gidx(gidxA, part)
        fire_gathers(gidxA, rowsA, sgA)
        pltpu.async_copy(gidx_src(part, 1), gidxB, siB)

        def phase(g, cur, nxt, first=False, fire_next=True, load_next=True):
            cg, cr, csg, css, csi = cur
            ng, nr, nsg, nss, nsi = nxt
            if not first:
                wait_scatters(nr, g - 1, nss)
            if fire_next:
                pltpu.make_async_copy(gidx_src(part, g + 1), ng, nsi).wait()
                to_gidx(ng, part)
                fire_gathers(ng, nr, nsg)
            wait_gathers(cg, cr, csg)
            if load_next:
                pltpu.async_copy(gidx_src(part, g + 2), cg, csi)
            fire_scatters(cr, g, css)

        phase(0, sets[0], sets[1], first=True)

        def body(i, carry):
            g = 1 + 2 * i
            phase(g, sets[1], sets[0])
            phase(g + 1, sets[0], sets[1])
            return carry

        lax.fori_loop(0, (NGRP - 3) // 2, body, 0)
        phase(NGRP - 2, sets[1], sets[0], load_next=False)
        phase(NGRP - 1, sets[0], sets[1], fire_next=False, load_next=False)
        wait_scatters(rowsA, NGRP - 1, ssA)

        plsc.subcore_barrier()
        pltpu.sync_copy(acc.at[pl.ds(row0, RPT)],
                        out_hbm.at[pl.ds(row0, RPT), part])
        plsc.subcore_barrier()


BR = 2000
GRID = N // BR


def _full(shape):
    return pl.BlockSpec(shape, lambda i: tuple(0 for _ in shape))


def _rows(shape):
    return pl.BlockSpec(shape, lambda i: (i,) + tuple(0 for _ in shape[1:]))


def _with_dis(u, dis):
    pad = jnp.zeros((u.shape[0], LANES - H - 1), jnp.float32)
    return jnp.concatenate([u, dis, pad], axis=1)


def _embed_pre1_body(dx, cx, degp, wbd, bc, wg0, bg0, wg1, xdc_o, u1_o):
    xd = dx[:, 6:32]
    xc = jnp.dot(cx[...], wbd[...], preferred_element_type=jnp.float32) + bc[...]
    xdc = jnp.concatenate([xd, xc], axis=1)
    xdc_o[...] = xdc
    deg = degp[:, 0] + degp[:, W16] + 1.0
    dis = lax.rsqrt(deg)[:, None]
    xg0 = _leaky(jnp.dot(xdc, wg0[...], preferred_element_type=jnp.float32)
                 + bg0[...])
    u1 = jnp.dot(xg0, wg1[...], preferred_element_type=jnp.float32) * dis
    u1_o[...] = _with_dis(u1, dis)


def _embed_pre1(dx, cx, degp, wbd, bc, wg0, bg0, wg1):
    return pl.pallas_call(
        _embed_pre1_body,
        grid=(GRID,),
        in_specs=[
            _rows((BR, 32)),
            _rows((BR, 48)),
            _rows((BR, LANES)),
            _full((48, 12)),
            _full((1, 12)),
            _full((38, H)),
            _full((1, H)),
            _full((H, H)),
        ],
        out_specs=[_rows((BR, 38)), _rows((BR, LANES))],
        out_shape=[
            jax.ShapeDtypeStruct((N, 38), jnp.float32),
            jax.ShapeDtypeStruct((N, LANES), jnp.float32),
        ],
    )(dx, cx, degp, wbd, bc, wg0, bg0, wg1)


def _pre2_body(acc1, u1d, b1, wg2, u2_o):
    d = u1d[:, H:H + 1]
    xg1 = _leaky(d * (acc1[:, :H] + u1d[:, :H]) + b1[...])
    u2 = jnp.dot(xg1, wg2[...], preferred_element_type=jnp.float32) * d
    u2_o[...] = _with_dis(u2, d)


def _pre2(acc1, u1d, b1, wg2):
    return pl.pallas_call(
        _pre2_body,
        grid=(GRID,),
        in_specs=[
            _rows((BR, LANES)),
            _rows((BR, LANES)),
            _full((1, H)),
            _full((H, H)),
        ],
        out_specs=_rows((BR, LANES)),
        out_shape=jax.ShapeDtypeStruct((N, LANES), jnp.float32),
    )(acc1, u1d, b1, wg2)


def _head_body(acc2, u2d, b2, xdc, wd1, bd1, wd2, bd2, wcr, bcr,
               wp1, bp1, wp2, bp2, out_o):
    d = u2d[:, H:H + 1]
    xg2 = _leaky(d * (acc2[:, :H] + u2d[:, :H]) + b2[...])
    x = jnp.concatenate([xdc[...], xg2], axis=1)
    h = _leaky(jnp.dot(x, wd1[...], preferred_element_type=jnp.float32)
               + bd1[...])
    deep = _leaky(jnp.dot(h, wd2[...], preferred_element_type=jnp.float32)
                  + bd2[...])
    xl = x
    for i in range(2):
        s = jnp.sum(xl * wcr[i:i + 1, :], axis=1, keepdims=True)
        xl = x * s + bcr[i:i + 1, :] + xl
    xc2 = jnp.concatenate([deep, xl], axis=1)
    p1 = _leaky(jnp.dot(xc2, wp1[...], preferred_element_type=jnp.float32)
                + bp1[...])
    p2 = jnp.dot(p1, wp2[...], preferred_element_type=jnp.float32) + bp2[...]
    out_o[...] = jax.nn.sigmoid(p2)


def _head(acc2, u2d, b2, xdc, wd1, bd1, wd2, bd2, wcr, bcr,
          wp1, bp1, wp2, bp2):
    NH1 = 134
    return pl.pallas_call(
        _head_body,
        grid=(GRID,),
        in_specs=[
            _rows((BR, LANES)),
            _rows((BR, LANES)),
            _full((1, H)),
            _rows((BR, 38)),
            _full((NH1, NH1)),
            _full((1, NH1)),
            _full((NH1, NH1)),
            _full((1, NH1)),
            _full((2, NH1)),
            _full((2, NH1)),
            _full((2 * NH1, NH1)),
            _full((1, NH1)),
            _full((NH1, 1)),
            _full((1, 1)),
        ],
        out_specs=_rows((BR, 1)),
        out_shape=jax.ShapeDtypeStruct((N, 1), jnp.float32),
    )(acc2, u2d, b2, xdc, wd1, bd1, wd2, bd2, wcr, bcr,
      wp1, bp1, wp2, bp2)


def kernel(discrete_x, continous_x, edge_index, edge_attr, churn_date,
           W_c, b_c, W_g0, b_g0, W_gcn1, b_gcn1, W_gcn2, b_gcn2,
           W_d1, b_d1, W_d2, b_d2, w_cross, b_cross, W_p1, b_p1, W_p2, b_p2):
    src = edge_index[0]
    dst = edge_index[1]
    zeros = jnp.zeros((RPT, W16), jnp.float32)
    ones = jnp.ones((DEG_CHUNK, W16), jnp.float32)

    degp = _deg_kernel(dst.reshape(E // DEG_CHUNK, DEG_CHUNK),
                       ones, zeros).reshape(NPAD, LANES)

    wbd = jnp.zeros((48, 12), jnp.float32)
    for g in range(3):
        wbd = wbd.at[g * 16:(g + 1) * 16, g * 4:(g + 1) * 4].set(W_c)
    bc = jnp.tile(b_c, 3)[None, :]

    dst2d = dst.reshape(E // AGG_CHUNK, AGG_CHUNK)
    xdc, u1d = _embed_pre1(discrete_x, continous_x, degp, wbd, bc,
                           W_g0, b_g0[None, :], W_gcn1)
    acc1 = _agg_kernel(u1d.reshape(N * GROUPS, W16), src,
                       dst2d, zeros).reshape(NPAD, LANES)
    u2d = _pre2(acc1, u1d, b_gcn1[None, :], W_gcn2)
    acc2 = _agg_kernel(u2d.reshape(N * GROUPS, W16), src,
                       dst2d, zeros).reshape(NPAD, LANES)
    return _head(acc2, u2d, b_gcn2[None, :], xdc,
                 W_d1, b_d1[None, :], W_d2, b_d2[None, :],
                 w_cross, b_cross, W_p1, b_p1[None, :], W_p2, b_p2[None, :])


# speedup vs baseline: 16.4004x; 1.1418x over previous
import functools

import jax
import jax.numpy as jnp
from jax import lax
from jax.experimental import pallas as pl
from jax.experimental.pallas import tpu as pltpu
from jax.experimental.pallas import tpu_sc as plsc

N = 50000
E = 800000
H = 96
P = 6
W16 = 16
LANES = 128
GROUPS = LANES // W16
TILES = 16
NCORES = 2
NPAD = 51200
RPT = NPAD // TILES
PARTS_PER_CORE = P // NCORES

AGG_EDGES_PER_TILE = E // TILES
AGG_CHUNK = 80
AGG_CROWS = AGG_EDGES_PER_TILE // AGG_CHUNK
GK = 25
NGRP = AGG_CROWS // GK

DEG_EDGES_PER_TILE = E // (NCORES * TILES)
DEG_CHUNK = 100
DEG_CROWS = DEG_EDGES_PER_TILE // DEG_CHUNK
DEG_FPB = 10

_mesh = plsc.VectorSubcoreMesh(core_axis_name="c", subcore_axis_name="s",
                               num_cores=NCORES, num_subcores=TILES)
_sc_params = pltpu.CompilerParams(use_tc_tiling_on_sc=False)


def _leaky(x):
    return jnp.where(x > 0, x, 0.01 * x)


@functools.partial(
    pl.kernel,
    out_type=jax.ShapeDtypeStruct((NPAD, GROUPS, W16), jnp.float32),
    mesh=_mesh,
    scratch_types=[
        pltpu.VMEM((DEG_CROWS, DEG_CHUNK), jnp.int32),
        pltpu.VMEM((DEG_CHUNK, W16), jnp.float32),
        pltpu.VMEM_SHARED((NPAD, W16), jnp.float32),
        pltpu.SemaphoreType.DMA,
    ],
    compiler_params=_sc_params,
)
def _deg_kernel(dst2d_hbm, ones_hbm, zeros_hbm, out_hbm, didx, ones_v, acc,
                sem):
    sc = lax.axis_index("c")
    sub = lax.axis_index("s")
    row0 = sub * RPT
    crow0 = (sc * TILES + sub) * DEG_CROWS
    pltpu.sync_copy(dst2d_hbm.at[pl.ds(crow0, DEG_CROWS)], didx)
    pltpu.sync_copy(zeros_hbm, acc.at[pl.ds(row0, RPT)])
    pltpu.sync_copy(ones_hbm, ones_v)
    plsc.subcore_barrier()

    def fire(i, carry):
        for k in range(DEG_FPB):
            pltpu.async_copy(ones_v, acc.at[didx.at[i * DEG_FPB + k]], sem,
                             add=True)
        return carry

    def drain(i, carry):
        for k in range(DEG_FPB):
            pltpu.make_async_copy(
                ones_v, acc.at[didx.at[i * DEG_FPB + k]], sem).wait()
        return carry

    lax.fori_loop(0, DEG_CROWS // DEG_FPB, fire, 0)
    lax.fori_loop(0, DEG_CROWS // DEG_FPB, drain, 0)
    plsc.subcore_barrier()
    pltpu.sync_copy(acc.at[pl.ds(row0, RPT)],
                    out_hbm.at[pl.ds(row0, RPT), sc])


@functools.partial(
    pl.kernel,
    out_type=jax.ShapeDtypeStruct((NPAD, GROUPS, W16), jnp.float32),
    mesh=_mesh,
    scratch_types=[
        pltpu.VMEM((GK, AGG_CHUNK), jnp.int32),
        pltpu.VMEM((GK, AGG_CHUNK), jnp.int32),
        pltpu.VMEM((GK * AGG_CHUNK,), jnp.int32),
        pltpu.VMEM((GK * AGG_CHUNK,), jnp.int32),
        pltpu.VMEM((GK, AGG_CHUNK, W16), jnp.float32),
        pltpu.VMEM((GK, AGG_CHUNK, W16), jnp.float32),
        pltpu.VMEM_SHARED((NPAD, W16), jnp.float32),
        pltpu.SemaphoreType.DMA,
        pltpu.SemaphoreType.DMA,
        pltpu.SemaphoreType.DMA,
        pltpu.SemaphoreType.DMA,
        pltpu.SemaphoreType.DMA,
        pltpu.SemaphoreType.DMA,
        pltpu.SemaphoreType.DMA,
        pltpu.SemaphoreType.DMA,
    ],
    compiler_params=_sc_params,
)
def _agg_kernel(utab_hbm, src_hbm, dst2d_hbm, zeros_hbm, out_hbm,
                didxA, didxB, gidxA, gidxB, rowsA, rowsB, acc,
                sgA, sgB, ssA, ssB, siA, siB, sdA, sdB):
    sc = lax.axis_index("c")
    sub = lax.axis_index("s")
    row0 = sub * RPT
    crow0 = sub * AGG_CROWS

    def to_gidx(gidx, part):
        def body(j, carry):
            s = gidx[pl.ds(j * W16, W16)]
            gidx[pl.ds(j * W16, W16)] = s * GROUPS + part
            return carry
        lax.fori_loop(0, GK * AGG_CHUNK // W16, body, 0)

    def fire_gathers(gidx, rows, sem):
        def body(k, carry):
            pltpu.async_copy(
                utab_hbm.at[gidx.at[pl.ds(k * AGG_CHUNK, AGG_CHUNK)]],
                rows.at[k], sem)
            return carry
        lax.fori_loop(0, GK, body, 0)

    def wait_gathers(gidx, rows, sem):
        def body(k, carry):
            pltpu.make_async_copy(
                utab_hbm.at[gidx.at[pl.ds(k * AGG_CHUNK, AGG_CHUNK)]],
                rows.at[k], sem).wait()
            return carry
        lax.fori_loop(0, GK, body, 0)

    def fire_scatters(rows, didx, sem):
        def body(k, carry):
            pltpu.async_copy(rows.at[k], acc.at[didx.at[k]], sem, add=True)
            return carry
        lax.fori_loop(0, GK, body, 0)

    def wait_scatters(rows, didx, sem):
        def body(k, carry):
            pltpu.make_async_copy(rows.at[k], acc.at[didx.at[k]], sem).wait()
            return carry
        lax.fori_loop(0, GK, body, 0)

    def gidx_src(g):
        return src_hbm.at[pl.ds((crow0 + g * GK) * AGG_CHUNK, GK * AGG_CHUNK)]

    def didx_src(g):
        return dst2d_hbm.at[pl.ds(crow0 + g * GK, GK)]

    for pp in range(PARTS_PER_CORE):
        part = sc * PARTS_PER_CORE + pp
        pltpu.sync_copy(zeros_hbm, acc.at[pl.ds(row0, RPT)])
        plsc.subcore_barrier()

        sets = ((didxA, gidxA, rowsA, sgA, ssA, siA, sdA),
                (didxB, gidxB, rowsB, sgB, ssB, siB, sdB))

        pltpu.sync_copy(didx_src(0), didxA)
        pltpu.sync_copy(gidx_src(0), gidxA)
        to_gidx(gidxA, part)
        fire_gathers(gidxA, rowsA, sgA)
        pltpu.async_copy(gidx_src(1), gidxB, siB)
        pltpu.async_copy(didx_src(1), didxB, sdB)

        def phase(g, cur, nxt, first=False, fire_next=True, load_next=True,
                  load_didx=True):
            cd, cg, cr, csg, css, csi, csd = cur
            nd, ng, nr, nsg, nss, nsi, nsd = nxt
            if not first:
                wait_scatters(nr, nd, nss)
                if load_didx:
                    pltpu.async_copy(didx_src(g + 1), nd, nsd)
            if fire_next:
                pltpu.make_async_copy(gidx_src(g + 1), ng, nsi).wait()
                to_gidx(ng, part)
                fire_gathers(ng, nr, nsg)
            wait_gathers(cg, cr, csg)
            if load_next:
                pltpu.async_copy(gidx_src(g + 2), cg, csi)
            if not first:
                pltpu.make_async_copy(didx_src(g), cd, csd).wait()
            fire_scatters(cr, cd, css)

        phase(0, sets[0], sets[1], first=True)

        def body(i, carry):
            g = 1 + 2 * i
            phase(g, sets[1], sets[0])
            phase(g + 1, sets[0], sets[1])
            return carry

        lax.fori_loop(0, (NGRP - 3) // 2, body, 0)
        phase(NGRP - 2, sets[1], sets[0], load_next=False)
        phase(NGRP - 1, sets[0], sets[1], fire_next=False, load_next=False,
              load_didx=False)
        wait_scatters(rowsA, didxA, ssA)

        plsc.subcore_barrier()
        pltpu.sync_copy(acc.at[pl.ds(row0, RPT)],
                        out_hbm.at[pl.ds(row0, RPT), part])
        plsc.subcore_barrier()


BR = 2000
GRID = N // BR


def _full(shape):
    return pl.BlockSpec(shape, lambda i: tuple(0 for _ in shape))


def _rows(shape):
    return pl.BlockSpec(shape, lambda i: (i,) + tuple(0 for _ in shape[1:]))


def _with_dis(u, dis):
    pad = jnp.zeros((u.shape[0], LANES - H - 1), jnp.float32)
    return jnp.concatenate([u, dis, pad], axis=1)


def _embed_pre1_body(dx, cx, degp, wbd, bc, wg0, bg0, wg1, xdc_o, u1_o):
    xd = dx[:, 6:32]
    xc = jnp.dot(cx[...], wbd[...], preferred_element_type=jnp.float32) + bc[...]
    xdc = jnp.concatenate([xd, xc], axis=1)
    xdc_o[...] = xdc
    deg = degp[:, 0] + degp[:, W16] + 1.0
    dis = lax.rsqrt(deg)[:, None]
    xg0 = _leaky(jnp.dot(xdc, wg0[...], preferred_element_type=jnp.float32)
                 + bg0[...])
    u1 = jnp.dot(xg0, wg1[...], preferred_element_type=jnp.float32) * dis
    u1_o[...] = _with_dis(u1, dis)


def _embed_pre1(dx, cx, degp, wbd, bc, wg0, bg0, wg1):
    return pl.pallas_call(
        _embed_pre1_body,
        grid=(GRID,),
        in_specs=[
            _rows((BR, 32)),
            _rows((BR, 48)),
            _rows((BR, LANES)),
            _full((48, 12)),
            _full((1, 12)),
            _full((38, H)),
            _full((1, H)),
            _full((H, H)),
        ],
        out_specs=[_rows((BR, 38)), _rows((BR, LANES))],
        out_shape=[
            jax.ShapeDtypeStruct((N, 38), jnp.float32),
            jax.ShapeDtypeStruct((N, LANES), jnp.float32),
        ],
    )(dx, cx, degp, wbd, bc, wg0, bg0, wg1)


def _pre2_body(acc1, u1d, b1, wg2, u2_o):
    d = u1d[:, H:H + 1]
    xg1 = _leaky(d * (acc1[:, :H] + u1d[:, :H]) + b1[...])
    u2 = jnp.dot(xg1, wg2[...], preferred_element_type=jnp.float32) * d
    u2_o[...] = _with_dis(u2, d)


def _pre2(acc1, u1d, b1, wg2):
    return pl.pallas_call(
        _pre2_body,
        grid=(GRID,),
        in_specs=[
            _rows((BR, LANES)),
            _rows((BR, LANES)),
            _full((1, H)),
            _full((H, H)),
        ],
        out_specs=_rows((BR, LANES)),
        out_shape=jax.ShapeDtypeStruct((N, LANES), jnp.float32),
    )(acc1, u1d, b1, wg2)


def _head_body(acc2, u2d, b2, xdc, wd1, bd1, wd2, bd2, wcr, bcr,
               wp1, bp1, wp2, bp2, out_o):
    d = u2d[:, H:H + 1]
    xg2 = _leaky(d * (acc2[:, :H] + u2d[:, :H]) + b2[...])
    x = jnp.concatenate([xdc[...], xg2], axis=1)
    h = _leaky(jnp.dot(x, wd1[...], preferred_element_type=jnp.float32)
               + bd1[...])
    deep = _leaky(jnp.dot(h, wd2[...], preferred_element_type=jnp.float32)
                  + bd2[...])
    xl = x
    for i in range(2):
        s = jnp.sum(xl * wcr[i:i + 1, :], axis=1, keepdims=True)
        xl = x * s + bcr[i:i + 1, :] + xl
    xc2 = jnp.concatenate([deep, xl], axis=1)
    p1 = _leaky(jnp.dot(xc2, wp1[...], preferred_element_type=jnp.float32)
                + bp1[...])
    p2 = jnp.dot(p1, wp2[...], preferred_element_type=jnp.float32) + bp2[...]
    out_o[...] = jax.nn.sigmoid(p2)


def _head(acc2, u2d, b2, xdc, wd1, bd1, wd2, bd2, wcr, bcr,
          wp1, bp1, wp2, bp2):
    NH1 = 134
    return pl.pallas_call(
        _head_body,
        grid=(GRID,),
        in_specs=[
            _rows((BR, LANES)),
            _rows((BR, LANES)),
            _full((1, H)),
            _rows((BR, 38)),
            _full((NH1, NH1)),
            _full((1, NH1)),
            _full((NH1, NH1)),
            _full((1, NH1)),
            _full((2, NH1)),
            _full((2, NH1)),
            _full((2 * NH1, NH1)),
            _full((1, NH1)),
            _full((NH1, 1)),
            _full((1, 1)),
        ],
        out_specs=_rows((BR, 1)),
        out_shape=jax.ShapeDtypeStruct((N, 1), jnp.float32),
    )(acc2, u2d, b2, xdc, wd1, bd1, wd2, bd2, wcr, bcr,
      wp1, bp1, wp2, bp2)


def kernel(discrete_x, continous_x, edge_index, edge_attr, churn_date,
           W_c, b_c, W_g0, b_g0, W_gcn1, b_gcn1, W_gcn2, b_gcn2,
           W_d1, b_d1, W_d2, b_d2, w_cross, b_cross, W_p1, b_p1, W_p2, b_p2):
    src = edge_index[0]
    dst = edge_index[1]
    zeros = jnp.zeros((RPT, W16), jnp.float32)
    ones = jnp.ones((DEG_CHUNK, W16), jnp.float32)

    degp = _deg_kernel(dst.reshape(E // DEG_CHUNK, DEG_CHUNK),
                       ones, zeros).reshape(NPAD, LANES)

    wbd = jnp.zeros((48, 12), jnp.float32)
    for g in range(3):
        wbd = wbd.at[g * 16:(g + 1) * 16, g * 4:(g + 1) * 4].set(W_c)
    bc = jnp.tile(b_c, 3)[None, :]

    dst2d = dst.reshape(E // AGG_CHUNK, AGG_CHUNK)
    xdc, u1d = _embed_pre1(discrete_x, continous_x, degp, wbd, bc,
                           W_g0, b_g0[None, :], W_gcn1)
    acc1 = _agg_kernel(u1d.reshape(N * GROUPS, W16), src,
                       dst2d, zeros).reshape(NPAD, LANES)
    u2d = _pre2(acc1, u1d, b_gcn1[None, :], W_gcn2)
    acc2 = _agg_kernel(u2d.reshape(N * GROUPS, W16), src,
                       dst2d, zeros).reshape(NPAD, LANES)
    return _head(acc2, u2d, b_gcn2[None, :], xdc,
                 W_d1, b_d1[None, :], W_d2, b_d2[None, :],
                 w_cross, b_cross, W_p1, b_p1[None, :], W_p2, b_p2[None, :])
